# Initial kernel scaffold; baseline (speedup 1.0000x reference)
import jax
import jax.numpy as jnp
from jax.experimental import pallas as pl


def kernel(x, edge_index, W1_rel, W1_root, b1, W2_rel, W2_root, b2, W3, a_src, a_dst, b3, Wres, bres, Wfc, bfc):
    raise NotImplementedError("write your pallas kernel here")


# speedup vs baseline: 78.0054x; 78.0054x over previous
import jax
import jax.numpy as jnp
from jax.experimental import pallas as pl

N = 256
E = 65280
EPAD = 65536
CHUNK = 2048
NCHUNK = EPAD // CHUNK
IN = 7
HID = 128
HEADS = 2
OUT = N * 6
FC_BLK = 2048


def _graph_body(edges_ref, x_ref, w1r_ref, w1t_ref, b1_ref, w2r_ref, w2t_ref,
                b2_ref, w3_ref, asrc_ref, adst_ref, b3_ref, wres_ref, bres_ref,
                h3_ref):
    f32 = jnp.float32
    iota_col = jax.lax.broadcasted_iota(jnp.int32, (N, 1), 0)

    def body(i, C):
        blk = edges_ref[:, pl.ds(i * CHUNK, CHUNK)]
        srow = blk[0:1, :]
        drow = blk[1:2, :]
        onehot_sT = (iota_col == srow).astype(jnp.bfloat16)
        onehot_dT = (iota_col == drow).astype(jnp.bfloat16)
        return C + jax.lax.dot_general(
            onehot_dT, onehot_sT, (((1,), (1,)), ((), ())),
            preferred_element_type=f32)

    C = jax.lax.fori_loop(0, NCHUNK, body, jnp.zeros((N, N), f32))

    x = x_ref[...]
    agg1 = jnp.dot(C, x, preferred_element_type=f32,
        precision=jax.lax.Precision.HIGHEST)
    h1 = (jnp.dot(agg1, w1r_ref[...], preferred_element_type=f32)
          + jnp.dot(x, w1t_ref[...], preferred_element_type=f32)
          + b1_ref[...]
          + jnp.dot(x, wres_ref[...], preferred_element_type=f32)
          + bres_ref[...])
    h1 = jnp.maximum(h1, 0.0)

    agg2 = jnp.dot(C, h1, preferred_element_type=f32,
        precision=jax.lax.Precision.HIGHEST)
    h2 = (jnp.dot(agg2, w2r_ref[...], preferred_element_type=f32)
          + jnp.dot(h1, w2t_ref[...], preferred_element_type=f32)
          + b2_ref[...])
    h2 = jnp.maximum(h2, 0.0)

    xp = jnp.dot(h2, w3_ref[...], preferred_element_type=f32)
    eye = (iota_col == jax.lax.broadcasted_iota(jnp.int32, (1, N), 1)
           ).astype(f32)
    Cp = C + eye
    mask = Cp > 0.0

    acc = None
    for h in range(HEADS):
        xph = xp[:, h * HID:(h + 1) * HID]
        a_s = asrc_ref[h:h + 1, :]
        a_d = adst_ref[h:h + 1, :]
        alpha_s_row = jax.lax.dot_general(
            a_s, xph, (((1,), (1,)), ((), ())), preferred_element_type=f32,
        precision=jax.lax.Precision.HIGHEST)
        alpha_d_col = jnp.sum(xph * a_d, axis=1, keepdims=True)
        e = alpha_d_col + alpha_s_row
        e = jnp.where(e >= 0.0, e, 0.2 * e)
        em = jnp.where(mask, e, -1e30)
        m = jnp.max(em, axis=1, keepdims=True)
        p = Cp * jnp.exp(em - m)
        denom = jnp.sum(p, axis=1, keepdims=True)
        o = jnp.dot(p / denom, xph, preferred_element_type=f32,
        precision=jax.lax.Precision.HIGHEST)
        acc = o if acc is None else acc + o

    h3 = jnp.maximum(acc * (1.0 / HEADS) + b3_ref[...], 0.0)
    h3_ref[...] = h3


def _fc_body(flat_ref, wfc_ref, bfc_ref, out_ref):
    i = pl.program_id(0)
    part = jnp.dot(flat_ref[...], wfc_ref[...],
                   preferred_element_type=jnp.float32)

    @pl.when(i == 0)
    def _():
        out_ref[...] = bfc_ref[...] + part

    @pl.when(i > 0)
    def _():
        out_ref[...] += part


def kernel(x, edge_index, W1_rel, W1_root, b1, W2_rel, W2_root, b2, W3,
           a_src, a_dst, b3, Wres, bres, Wfc, bfc):
    edges = jnp.pad(edge_index, ((0, 0), (0, EPAD - E)), constant_values=-1)
    h3 = pl.pallas_call(
        _graph_body,
        out_shape=jax.ShapeDtypeStruct((N, HID), jnp.float32),
    )(edges, x, W1_rel, W1_root, b1.reshape(1, HID), W2_rel, W2_root,
      b2.reshape(1, HID), W3, a_src, a_dst, b3.reshape(1, HID), Wres,
      bres.reshape(1, HID))

    flat = h3.reshape(1, N * HID)
    out = pl.pallas_call(
        _fc_body,
        grid=(N * HID // FC_BLK,),
        in_specs=[
            pl.BlockSpec((1, FC_BLK), lambda i: (0, i)),
            pl.BlockSpec((FC_BLK, OUT), lambda i: (i, 0)),
            pl.BlockSpec((1, OUT), lambda i: (0, 0)),
        ],
        out_specs=pl.BlockSpec((1, OUT), lambda i: (0, 0)),
        out_shape=jax.ShapeDtypeStruct((1, OUT), jnp.float32),
    )(flat, Wfc, bfc.reshape(1, OUT))
    return out
